# baseline (speedup 1.0000x reference)
import functools

import jax
import jax.numpy as jnp
from jax import lax
from jax.experimental import pallas as pl
from jax.experimental.pallas import tpu as pltpu
from jax.experimental.pallas import tpu_sc as plsc

_BATCH = 128
_DATA_DIM = 4096
_LANES = 16


def _make_sc_masked_identity(num_cores, num_subcores):
    num_workers = num_cores * num_subcores
    rows_per_worker = _BATCH // num_workers
    chunks = (rows_per_worker * _DATA_DIM) // _LANES
    mesh = plsc.VectorSubcoreMesh(core_axis_name="c", subcore_axis_name="s")

    @functools.partial(
        pl.kernel,
        out_type=jax.ShapeDtypeStruct((_BATCH, _DATA_DIM), jnp.float32),
        mesh=mesh,
        scratch_types=[pltpu.VMEM((rows_per_worker, _DATA_DIM), jnp.float32)],
    )
    def sc_kernel(x_hbm, out_hbm, buf):
        wid = lax.axis_index("s") * num_cores + lax.axis_index("c")
        base = wid * rows_per_worker
        pltpu.sync_copy(x_hbm.at[pl.ds(base, rows_per_worker)], buf)

        def chunk(i, carry):
            for r in range(rows_per_worker):
                sl = pl.ds(i * _LANES, _LANES)
                v = buf[r, sl]
                buf[r, sl] = jnp.where(v != 0.0, v, 0.0)
            return carry

        lax.fori_loop(0, chunks // rows_per_worker, chunk, 0)
        pltpu.sync_copy(buf, out_hbm.at[pl.ds(base, rows_per_worker)])

    return sc_kernel


def kernel(t, x, embed_table):
    info = plsc.get_sparse_core_info()
    sc = _make_sc_masked_identity(info.num_cores, info.num_subcores)
    return sc(x)


# speedup vs baseline: 1.0264x; 1.0264x over previous
import functools

import jax
import jax.numpy as jnp
from jax import lax
from jax.experimental import pallas as pl
from jax.experimental.pallas import tpu as pltpu
from jax.experimental.pallas import tpu_sc as plsc

_BATCH = 128
_DATA_DIM = 4096
_LANES = 16


def _make_sc_masked_identity(num_cores, num_subcores):
    num_workers = num_cores * num_subcores
    rows_per_worker = _BATCH // num_workers
    mesh = plsc.VectorSubcoreMesh(core_axis_name="c", subcore_axis_name="s")

    @functools.partial(
        pl.kernel,
        out_type=jax.ShapeDtypeStruct((_BATCH, _DATA_DIM), jnp.float32),
        mesh=mesh,
        scratch_types=[
            pltpu.VMEM((rows_per_worker, _DATA_DIM), jnp.float32),
            pltpu.SemaphoreType.DMA,
            pltpu.SemaphoreType.DMA,
        ],
    )
    def sc_kernel(x_hbm, out_hbm, buf, sem_in, sem_out):
        wid = lax.axis_index("s") * num_cores + lax.axis_index("c")
        base = wid * rows_per_worker

        loads = [
            pltpu.async_copy(
                x_hbm.at[pl.ds(base + r, 1)], buf.at[pl.ds(r, 1)], sem_in
            )
            for r in range(rows_per_worker)
        ]
        stores = []
        for r in range(rows_per_worker):
            loads[r].wait()

            @plsc.parallel_loop(0, _DATA_DIM, step=_LANES, unroll=8)
            def _select(i, r=r):
                v = buf[r, pl.ds(i, _LANES)]
                buf[r, pl.ds(i, _LANES)] = jnp.where(v != 0.0, v, 0.0)

            stores.append(
                pltpu.async_copy(
                    buf.at[pl.ds(r, 1)], out_hbm.at[pl.ds(base + r, 1)], sem_out
                )
            )
        for s in stores:
            s.wait()

    return sc_kernel


def kernel(t, x, embed_table):
    info = plsc.get_sparse_core_info()
    sc = _make_sc_masked_identity(info.num_cores, info.num_subcores)
    return sc(x)
